# baseline (speedup 1.0000x reference)
import jax
import jax.numpy as jnp
from jax.experimental import pallas as pl
from jax.experimental.pallas import tpu as pltpu

N = 4096
D = 64
BM = 512
NBLK = N // BM
NBUF = 6


def _spmm_kernel(inp_hbm, w_ref, out_ref, bufs, sems):
    def copy(i):
        return pltpu.make_async_copy(
            inp_hbm.at[pl.ds(i * BM, BM), :], bufs.at[i % NBUF],
            sems.at[i % NBUF])

    for i in range(NBUF):
        copy(i).start()
    for i in range(NBLK):
        copy(i).wait()
        out_ref[pl.ds(i * BM, BM), :] = jnp.dot(
            bufs[i % NBUF], w_ref[...], preferred_element_type=jnp.float32)
        if i + NBUF < NBLK:
            copy(i + NBUF).start()


@jax.jit
def kernel(inp, weight):
    return pl.pallas_call(
        _spmm_kernel,
        in_specs=[
            pl.BlockSpec(memory_space=pltpu.MemorySpace.HBM),
            pl.BlockSpec(memory_space=pltpu.MemorySpace.VMEM),
        ],
        out_specs=pl.BlockSpec(memory_space=pltpu.MemorySpace.VMEM),
        out_shape=jax.ShapeDtypeStruct((N, D), jnp.float32),
        scratch_shapes=[
            pltpu.VMEM((NBUF, BM, N), jnp.float32),
            pltpu.SemaphoreType.DMA((NBUF,)),
        ],
        compiler_params=pltpu.CompilerParams(
            skip_device_barrier=True,
            disable_bounds_checks=True,
        ),
    )(inp, weight)
